# baseline (speedup 1.0000x reference)
import functools

import jax
import jax.numpy as jnp
from jax import lax
from jax.experimental import pallas as pl
from jax.experimental.pallas import tpu as pltpu
from jax.experimental.pallas import tpu_sc as plsc

_NC = 2
_NS = 16
_LANES = 16


@functools.lru_cache(maxsize=None)
def _build(B, S, V, D):
  NW = _NC * _NS
  ROWS = B * S
  assert ROWS % NW == 0
  RPW = ROWS // NW
  IDMA = 128
  CHUNK = 512
  assert RPW % CHUNK == 0 and CHUNK % IDMA == 0
  NCHUNK = RPW // CHUNK
  NDMA = CHUNK // IDMA

  mesh = plsc.VectorSubcoreMesh(core_axis_name="c", subcore_axis_name="s")

  @functools.partial(
      pl.kernel,
      mesh=mesh,
      compiler_params=pltpu.CompilerParams(use_tc_tiling_on_sc=False),
      out_type=jax.ShapeDtypeStruct((ROWS, D), jnp.float32),
      scratch_types=[
          pltpu.VMEM((RPW,), jnp.int32),
          pltpu.VMEM((S, D), jnp.float32),
          pltpu.VMEM((CHUNK, D), jnp.float32),
          pltpu.SemaphoreType.DMA,
      ],
  )
  def emb(seq_hbm, tok_hbm, pos_hbm, out_hbm, idx_v, pos_v, rows_v, sem):
    wid = lax.axis_index("s") * _NC + lax.axis_index("c")
    rowbase = wid * RPW
    pltpu.sync_copy(pos_hbm, pos_v)
    pltpu.sync_copy(seq_hbm.at[pl.ds(rowbase, RPW)], idx_v)

    def chunk_body(g, carry):
      cbase = g * CHUNK
      handles = []
      for j in range(NDMA):
        handles.append(
            pltpu.async_copy(
                tok_hbm.at[idx_v.at[pl.ds(cbase + j * IDMA, IDMA)]],
                rows_v.at[pl.ds(j * IDMA, IDMA), :],
                sem,
            )
        )
      for h in handles:
        h.wait()

      p0 = lax.rem(cbase, S)

      def row_body(r, p):
        for j in range(D // _LANES):
          pv = pos_v[p, pl.ds(j * _LANES, _LANES)]
          plsc.addupdate(rows_v.at[r, pl.ds(j * _LANES, _LANES)], pv)
        p = p + 1
        return jnp.where(p == S, 0, p)

      lax.fori_loop(0, CHUNK, row_body, p0)

      pltpu.sync_copy(rows_v, out_hbm.at[pl.ds(rowbase + cbase, CHUNK), :])
      return carry

    lax.fori_loop(0, NCHUNK, chunk_body, 0)

  return emb


def kernel(seq, token_table, pos_table):
  B, S = seq.shape
  V, D = token_table.shape
  emb = _build(B, S, V, D)
  out = emb(seq.reshape(-1), token_table, pos_table)
  return out.reshape(B, S, D)


# speedup vs baseline: 1.2711x; 1.2711x over previous
import functools

import jax
import jax.numpy as jnp
from jax import lax
from jax.experimental import pallas as pl
from jax.experimental.pallas import tpu as pltpu
from jax.experimental.pallas import tpu_sc as plsc

_NC = 2
_NS = 16
_L = 16
_NBUF = 4


@functools.lru_cache(maxsize=None)
def _build(B, S, V, D):
  NW = _NC * _NS
  assert B % NW == 0
  BPW = B // NW
  NCHUNK = BPW
  assert NCHUNK % _NBUF == 0 and NCHUNK // _NBUF >= 3
  assert S == 200
  SPLITS = ((0, 128), (128, 72))

  mesh = plsc.VectorSubcoreMesh(core_axis_name="c", subcore_axis_name="s")

  @functools.partial(
      pl.kernel,
      mesh=mesh,
      compiler_params=pltpu.CompilerParams(use_tc_tiling_on_sc=False),
      out_type=jax.ShapeDtypeStruct((B, S, D), jnp.float32),
      scratch_types=[
          pltpu.VMEM((BPW, S), jnp.int32),
          pltpu.VMEM((S, D), jnp.float32),
          pltpu.VMEM((_NBUF, S, D), jnp.float32),
          pltpu.SemaphoreType.DMA,
          pltpu.SemaphoreType.DMA,
      ],
  )
  def emb(seq_hbm, tok_hbm, pos_hbm, out_hbm, idx_v, pos_v, rows_v, gsem,
          osem):
    wid = lax.axis_index("s") * _NC + lax.axis_index("c")
    b0 = wid * BPW
    pltpu.sync_copy(pos_hbm, pos_v)
    pltpu.sync_copy(seq_hbm.at[pl.ds(b0, BPW), :], idx_v)

    def issue_gather(g, buf):
      for off, n in SPLITS:
        pltpu.async_copy(
            tok_hbm.at[idx_v.at[g, pl.ds(off, n)]],
            rows_v.at[buf, pl.ds(off, n), :],
            gsem,
        )

    def wait_gather(buf):
      for off, n in SPLITS:
        pltpu.make_async_copy(
            tok_hbm.at[pl.ds(0, n), :],
            rows_v.at[buf, pl.ds(off, n), :],
            gsem,
        ).wait()

    def issue_out(g, buf):
      pltpu.async_copy(rows_v.at[buf], out_hbm.at[b0 + g], osem)

    def wait_out(buf):
      pltpu.make_async_copy(rows_v.at[buf], out_hbm.at[b0], osem).wait()

    def add_pos(buf):
      def row_body(s, carry):
        for j in range(D // _L):
          pv = pos_v[s, pl.ds(j * _L, _L)]
          plsc.addupdate(rows_v.at[buf, s, pl.ds(j * _L, _L)], pv)
        return carry

      lax.fori_loop(0, S, row_body, 0)

    def slot(g, b, *, first=False, last=False):
      wait_gather(b)
      if not first:
        wait_out((b + 3) % _NBUF)
      if not last:
        issue_gather(g + 3, (b + 3) % _NBUF)
      add_pos(b)
      issue_out(g, b)

    for g in range(3):
      issue_gather(g, g)

    slot(0, 0, first=True)
    for b in range(1, _NBUF):
      slot(b, b)

    def turn(g4, carry):
      for b in range(_NBUF):
        slot(g4 * _NBUF + b, b)
      return carry

    lax.fori_loop(1, NCHUNK // _NBUF - 1, turn, 0)

    gl = NCHUNK - _NBUF
    slot(gl, 0)
    for b in range(1, _NBUF):
      slot(gl + b, b, last=True)
    wait_out(_NBUF - 1)

  return emb


def kernel(seq, token_table, pos_table):
  B, S = seq.shape
  V, D = token_table.shape
  emb = _build(B, S, V, D)
  return emb(seq, token_table, pos_table)


# speedup vs baseline: 1.2713x; 1.0002x over previous
import functools

import jax
import jax.numpy as jnp
from jax import lax
from jax.experimental import pallas as pl
from jax.experimental.pallas import tpu as pltpu
from jax.experimental.pallas import tpu_sc as plsc

_NC = 2
_NS = 16
_L = 16
_NBUF = 4


@functools.lru_cache(maxsize=None)
def _build(B, S, V, D):
  NW = _NC * _NS
  assert B % NW == 0
  BPW = B // NW
  NCHUNK = BPW
  assert NCHUNK % _NBUF == 0 and NCHUNK // _NBUF >= 3
  assert S == 200
  SPLITS = ((0, 128), (128, 72))

  mesh = plsc.VectorSubcoreMesh(core_axis_name="c", subcore_axis_name="s")

  @functools.partial(
      pl.kernel,
      mesh=mesh,
      compiler_params=pltpu.CompilerParams(use_tc_tiling_on_sc=False),
      out_type=jax.ShapeDtypeStruct((B, S, D), jnp.float32),
      scratch_types=[
          pltpu.VMEM((BPW, S), jnp.float32),
          pltpu.VMEM((BPW, S), jnp.int32),
          pltpu.VMEM((S, D), jnp.float32),
          pltpu.VMEM((_NBUF, S, D), jnp.float32),
          pltpu.SemaphoreType.DMA,
          pltpu.SemaphoreType.DMA,
      ],
  )
  def emb(seq_hbm, tok_hbm, pos_hbm, out_hbm, idxf_v, idx_v, pos_v, rows_v,
          gsem, osem):
    wid = lax.axis_index("s") * _NC + lax.axis_index("c")
    b0 = wid * BPW
    pltpu.sync_copy(pos_hbm, pos_v)
    pltpu.sync_copy(seq_hbm.at[pl.ds(b0, BPW), :], idxf_v)

    COLS = tuple(j * _L for j in range(S // _L)) + (S - _L,)

    def conv_row(r, carry):
      for c in COLS:
        idx_v[r, pl.ds(c, _L)] = jax.lax.bitcast_convert_type(
            idxf_v[r, pl.ds(c, _L)], jnp.int32)
      return carry

    lax.fori_loop(0, BPW, conv_row, 0)

    def issue_gather(g, buf):
      for off, n in SPLITS:
        pltpu.async_copy(
            tok_hbm.at[idx_v.at[g, pl.ds(off, n)]],
            rows_v.at[buf, pl.ds(off, n), :],
            gsem,
        )

    def wait_gather(buf):
      for off, n in SPLITS:
        pltpu.make_async_copy(
            tok_hbm.at[pl.ds(0, n), :],
            rows_v.at[buf, pl.ds(off, n), :],
            gsem,
        ).wait()

    def issue_out(g, buf):
      pltpu.async_copy(rows_v.at[buf], out_hbm.at[b0 + g], osem)

    def wait_out(buf):
      pltpu.make_async_copy(rows_v.at[buf], out_hbm.at[b0], osem).wait()

    def add_pos(buf):
      def row_body(s, carry):
        for j in range(D // _L):
          pv = pos_v[s, pl.ds(j * _L, _L)]
          plsc.addupdate(rows_v.at[buf, s, pl.ds(j * _L, _L)], pv)
        return carry

      lax.fori_loop(0, S, row_body, 0)

    def slot(g, b, *, first=False, last=False):
      wait_gather(b)
      if not first:
        wait_out((b + 3) % _NBUF)
      if not last:
        issue_gather(g + 3, (b + 3) % _NBUF)
      add_pos(b)
      issue_out(g, b)

    for g in range(3):
      issue_gather(g, g)

    slot(0, 0, first=True)
    for b in range(1, _NBUF):
      slot(b, b)

    def turn(g4, carry):
      for b in range(_NBUF):
        slot(g4 * _NBUF + b, b)
      return carry

    lax.fori_loop(1, NCHUNK // _NBUF - 1, turn, 0)

    gl = NCHUNK - _NBUF
    slot(gl, 0)
    for b in range(1, _NBUF):
      slot(gl + b, b, last=True)
    wait_out(_NBUF - 1)

  return emb


def kernel(seq, token_table, pos_table):
  B, S = seq.shape
  V, D = token_table.shape
  emb = _build(B, S, V, D)
  seq_bits = jax.lax.bitcast_convert_type(seq, jnp.float32)
  return emb(seq_bits, token_table, pos_table)
